# baseline (speedup 1.0000x reference)
import functools

import jax
import jax.numpy as jnp
from jax.experimental import pallas as pl

_MASK_VAL = -1000000000000.0


def _head_kernel(feats_ref, wv_ref, bv_ref, wn_ref, bn_ref, mv_ref, mn_ref,
                 ov_ref, on_ref):
    x = feats_ref[...].astype(jnp.bfloat16)
    v = jnp.dot(x, wv_ref[...], preferred_element_type=jnp.float32)
    v = v + bv_ref[...]
    ov_ref[...] = jnp.where(mv_ref[...] != 0.0, v, _MASK_VAL)
    n = jnp.dot(x, wn_ref[...], preferred_element_type=jnp.float32)
    n = n + bn_ref[...]
    on_ref[...] = jnp.where(mn_ref[...] != 0.0, n, _MASK_VAL)


@functools.partial(jax.jit, static_argnames=("block_m",))
def _masked_head(feats, W_verb, b_verb, W_noun, b_noun,
                 seen_verb_mask, seen_noun_mask, block_m=512):
    batch, d_feat = feats.shape
    num_verbs = W_verb.shape[1]
    num_nouns = W_noun.shape[1]
    grid = (batch // block_m,)

    wv = W_verb.astype(jnp.bfloat16)
    wn = W_noun.astype(jnp.bfloat16)
    bv = b_verb.reshape(1, num_verbs)
    bn = b_noun.reshape(1, num_nouns)
    mv = seen_verb_mask.astype(jnp.float32).reshape(1, num_verbs)
    mn = seen_noun_mask.astype(jnp.float32).reshape(1, num_nouns)

    full = lambda *shape: pl.BlockSpec(shape, lambda i: (0,) * len(shape))
    return pl.pallas_call(
        _head_kernel,
        grid=grid,
        in_specs=[
            pl.BlockSpec((block_m, d_feat), lambda i: (i, 0)),
            full(d_feat, num_verbs),
            full(1, num_verbs),
            full(d_feat, num_nouns),
            full(1, num_nouns),
            full(1, num_verbs),
            full(1, num_nouns),
        ],
        out_specs=(
            pl.BlockSpec((block_m, num_verbs), lambda i: (i, 0)),
            pl.BlockSpec((block_m, num_nouns), lambda i: (i, 0)),
        ),
        out_shape=(
            jax.ShapeDtypeStruct((batch, num_verbs), jnp.float32),
            jax.ShapeDtypeStruct((batch, num_nouns), jnp.float32),
        ),
    )(feats, wv, bv, wn, bn, mv, mn)


def kernel(feats, W_verb, b_verb, W_noun, b_noun, seen_verb_mask, seen_noun_mask):
    return _masked_head(feats, W_verb, b_verb, W_noun, b_noun,
                        seen_verb_mask, seen_noun_mask)


# speedup vs baseline: 1.1618x; 1.1618x over previous
import functools

import jax
import jax.numpy as jnp
from jax.experimental import pallas as pl

_MASK_VAL = -1000000000000.0


def _head_kernel(feats_ref, wv_ref, bv_ref, wn_ref, bn_ref, mv_ref, mn_ref,
                 ov_ref, on_ref):
    x = feats_ref[...].astype(jnp.bfloat16)
    v = jnp.dot(x, wv_ref[...], preferred_element_type=jnp.float32)
    v = v + bv_ref[...]
    ov_ref[...] = jnp.where(mv_ref[...] != 0.0, v, _MASK_VAL)
    n = jnp.dot(x, wn_ref[...], preferred_element_type=jnp.float32)
    n = n + bn_ref[...]
    on_ref[...] = jnp.where(mn_ref[...] != 0.0, n, _MASK_VAL)


@functools.partial(jax.jit, static_argnames=("block_m",))
def _masked_head(feats, W_verb, b_verb, W_noun, b_noun,
                 seen_verb_mask, seen_noun_mask, block_m=2048):
    batch, d_feat = feats.shape
    num_verbs = W_verb.shape[1]
    num_nouns = W_noun.shape[1]
    grid = (batch // block_m,)

    wv = W_verb.astype(jnp.bfloat16)
    wn = W_noun.astype(jnp.bfloat16)
    bv = b_verb.reshape(1, num_verbs)
    bn = b_noun.reshape(1, num_nouns)
    mv = seen_verb_mask.astype(jnp.float32).reshape(1, num_verbs)
    mn = seen_noun_mask.astype(jnp.float32).reshape(1, num_nouns)

    full = lambda *shape: pl.BlockSpec(shape, lambda i: (0,) * len(shape))
    return pl.pallas_call(
        _head_kernel,
        grid=grid,
        in_specs=[
            pl.BlockSpec((block_m, d_feat), lambda i: (i, 0)),
            full(d_feat, num_verbs),
            full(1, num_verbs),
            full(d_feat, num_nouns),
            full(1, num_nouns),
            full(1, num_verbs),
            full(1, num_nouns),
        ],
        out_specs=(
            pl.BlockSpec((block_m, num_verbs), lambda i: (i, 0)),
            pl.BlockSpec((block_m, num_nouns), lambda i: (i, 0)),
        ),
        out_shape=(
            jax.ShapeDtypeStruct((batch, num_verbs), jnp.float32),
            jax.ShapeDtypeStruct((batch, num_nouns), jnp.float32),
        ),
    )(feats, wv, bv, wn, bn, mv, mn)


def kernel(feats, W_verb, b_verb, W_noun, b_noun, seen_verb_mask, seen_noun_mask):
    return _masked_head(feats, W_verb, b_verb, W_noun, b_noun,
                        seen_verb_mask, seen_noun_mask)


# speedup vs baseline: 2.0690x; 1.7808x over previous
import functools

import jax
import jax.numpy as jnp
from jax import lax
from jax.experimental import pallas as pl

_MASK_VAL = -1000000000000.0

_DOT_T = (((1,), (1,)), ((), ()))


def _head_kernel(feats_ref, wv_ref, bv_ref, wn_ref, bn_ref, mv_ref, mn_ref,
                 ov_ref, on_ref):
    x = feats_ref[...].astype(jnp.bfloat16)
    v = lax.dot_general(wv_ref[...], x, _DOT_T,
                        preferred_element_type=jnp.float32)
    v = v + bv_ref[...]
    ov_ref[...] = jnp.where(mv_ref[...] != 0.0, v, _MASK_VAL)
    n = lax.dot_general(wn_ref[...], x, _DOT_T,
                        preferred_element_type=jnp.float32)
    n = n + bn_ref[...]
    on_ref[...] = jnp.where(mn_ref[...] != 0.0, n, _MASK_VAL)


@functools.partial(jax.jit, static_argnames=("block_m",))
def _masked_head(feats, W_verb, b_verb, W_noun, b_noun,
                 seen_verb_mask, seen_noun_mask, block_m=2048):
    batch, d_feat = feats.shape
    num_verbs = W_verb.shape[1]
    num_nouns = W_noun.shape[1]
    grid = (batch // block_m,)

    wv = W_verb.T.astype(jnp.bfloat16)
    wn = W_noun.T.astype(jnp.bfloat16)
    bv = b_verb.reshape(num_verbs, 1)
    bn = b_noun.reshape(num_nouns, 1)
    mv = seen_verb_mask.astype(jnp.float32).reshape(num_verbs, 1)
    mn = seen_noun_mask.astype(jnp.float32).reshape(num_nouns, 1)

    full = lambda *shape: pl.BlockSpec(shape, lambda i: (0,) * len(shape))
    vt, nt = pl.pallas_call(
        _head_kernel,
        grid=grid,
        in_specs=[
            pl.BlockSpec((block_m, d_feat), lambda i: (i, 0)),
            full(num_verbs, d_feat),
            full(num_verbs, 1),
            full(num_nouns, d_feat),
            full(num_nouns, 1),
            full(num_verbs, 1),
            full(num_nouns, 1),
        ],
        out_specs=(
            pl.BlockSpec((num_verbs, block_m), lambda i: (0, i)),
            pl.BlockSpec((num_nouns, block_m), lambda i: (0, i)),
        ),
        out_shape=(
            jax.ShapeDtypeStruct((num_verbs, batch), jnp.float32),
            jax.ShapeDtypeStruct((num_nouns, batch), jnp.float32),
        ),
    )(feats, wv, bv, wn, bn, mv, mn)
    return vt.T, nt.T


def kernel(feats, W_verb, b_verb, W_noun, b_noun, seen_verb_mask, seen_noun_mask):
    return _masked_head(feats, W_verb, b_verb, W_noun, b_noun,
                        seen_verb_mask, seen_noun_mask)
